# Initial kernel scaffold; baseline (speedup 1.0000x reference)
import jax
import jax.numpy as jnp
from jax.experimental import pallas as pl


def kernel(x):
    raise NotImplementedError("write your pallas kernel here")


# speedup vs baseline: 8.6213x; 8.6213x over previous
import functools

import jax
import jax.numpy as jnp
from jax.experimental import pallas as pl
from jax.experimental.pallas import tpu as pltpu

K = 16
BIG = 3.0e38


def _normalize(x):
    ssq = jnp.sum(x * x, axis=0, keepdims=True)
    xn = x / jnp.maximum(jnp.sqrt(ssq), 1e-12)
    s = jnp.sum(xn * xn, axis=0, keepdims=True)
    return xn, s


def _knn_kernel(x_ref, xrow_ref, out_ref, *, rows: int, n: int, d: int):
    xn, s = _normalize(x_ref[0])
    xr, sr = _normalize(xrow_ref[0])
    p = jax.lax.dot_general(
        xr.astype(jnp.bfloat16), xn.astype(jnp.bfloat16),
        (((0,), (0,)), ((), ())),
        preferred_element_type=jnp.float32)
    dist = (jnp.transpose(sr) + (-2.0 * p)) + s

    lane = jax.lax.broadcasted_iota(jnp.int32, (rows, n), 1)
    for t in range(K):
        m = jnp.min(dist, axis=1, keepdims=True)
        cand = jnp.where(dist == m, lane, n)
        idx = jnp.min(cand, axis=1, keepdims=True)
        out_ref[0, :, t] = idx[:, 0]
        dist = jnp.where(lane == idx, BIG, dist)


def kernel(x):
    b, d, n, _ = x.shape
    xs = jnp.squeeze(x, axis=-1)
    rows = 256
    grid = (b, n // rows)
    nn_idx = pl.pallas_call(
        functools.partial(_knn_kernel, rows=rows, n=n, d=d),
        grid=grid,
        in_specs=[
            pl.BlockSpec((1, d, n), lambda bi, ri: (bi, 0, 0)),
            pl.BlockSpec((1, d, rows), lambda bi, ri: (bi, 0, ri)),
        ],
        out_specs=pl.BlockSpec((1, rows, K), lambda bi, ri: (bi, ri, 0)),
        out_shape=jax.ShapeDtypeStruct((b, n, K), jnp.int32),
    )(xs, xs)
    center_idx = jnp.broadcast_to(
        jnp.arange(n, dtype=jnp.int32)[None, :, None], (b, n, K))
    return jnp.stack((nn_idx, center_idx), axis=0)
